# baseline (speedup 1.0000x reference)
import functools

import jax
import jax.numpy as jnp
from jax import lax
from jax.experimental import pallas as pl
from jax.experimental.pallas import tpu as pltpu
from jax.experimental.pallas import tpu_sc as plsc

_NC = 2
_NT = 16
_L = 16

_splat_dnums = lax.GatherDimensionNumbers(
    offset_dims=(), collapsed_slice_dims=(0,), start_index_map=(0,))

_EC = 400
_NB = 3
_FC = 160
_ZC = 80


@functools.lru_cache(maxsize=None)
def _build(N: int, E: int, B: int):
    assert B == _NC * _L
    assert E % (_NT * _EC) == 0
    assert N % _FC == 0 and _FC % _ZC == 0
    ET = E // _NT
    NCH = ET // _EC
    NBLK = E // _EC
    G = _EC // _L
    NF = N // _FC
    FK = (NF + _NT - 1) // _NT
    FG = _FC // _L

    def body(src, edata, bias, out, t1, t2,
             acc, gath, ibuf, fbuf, zbuf, bbuf,
             lsem, gsem, ssem, fsem):
        c = lax.axis_index("c")
        s = lax.axis_index("s")
        bbase = c * NBLK + s * NCH
        coff = c * N

        def zfill(i, carry):
            zbuf[i, :] = jnp.zeros((_L,), jnp.float32)
            return carry
        lax.fori_loop(0, _ZC, zfill, 0)

        def zero_slice(r0):
            for z in range(_FC // _ZC):
                pltpu.sync_copy(zbuf, acc.at[pl.ds(r0 + z * _ZC, _ZC)])

        def spmm(src_hbm, vslot, dst_hbm, add_bias):
            def start_linear(i, p):
                pltpu.async_copy(edata.at[bbase + i], ibuf.at[p], lsem)

            def wait_linear(p):
                pltpu.make_async_copy(edata.at[0], ibuf.at[p], lsem).wait()

            def start_gather(p):
                pltpu.async_copy(src_hbm.at[ibuf.at[p, 0]], gath.at[p],
                                 gsem.at[p])

            def wait_gather(p):
                pltpu.make_async_copy(src_hbm.at[ibuf.at[p, 0]], gath.at[p],
                                      gsem.at[p]).wait()

            jsplat = [jnp.full((_L,), j, jnp.int32) for j in range(_L)]

            def multiply(p):
                def mul(g, carry):
                    b0 = g * _L
                    vv = plsc.bitcast(ibuf[p, vslot, pl.ds(b0, _L)],
                                      jnp.float32)
                    for j in range(_L):
                        sv = lax.gather(
                            vv, jsplat[j][:, None], _splat_dnums,
                            slice_sizes=(1,),
                            mode=lax.GatherScatterMode.PROMISE_IN_BOUNDS)
                        gath[p, b0 + j, :] = gath[p, b0 + j, :] * sv
                    return carry
                lax.fori_loop(0, G, mul, 0)

            def start_scatter(p):
                pltpu.async_copy(gath.at[p], acc.at[ibuf.at[p, 1]],
                                 ssem.at[p], add=True)

            def wait_scatter(p):
                pltpu.make_async_copy(
                    gath.at[p], acc.at[ibuf.at[p, 1]], ssem.at[p]).wait()

            start_linear(0, 0)
            wait_linear(0)
            start_gather(0)
            start_linear(1, 1 % _NB)

            def step(i, k, static_i=None):
                b = (k + 1) % _NB
                d = (k - 1) % _NB
                ii = i if static_i is None else static_i

                def guard(cond, fn):
                    if static_i is None:
                        pl.when(cond)(fn)
                    elif cond:
                        fn()

                def stage_next():
                    wait_linear(b)
                    start_gather(b)

                guard(ii + 1 < NCH if static_i is not None else i + 1 < NCH,
                      stage_next)
                wait_gather(k)
                multiply(k)
                guard(ii >= 1 if static_i is not None else i >= 1,
                      lambda: wait_scatter(d))
                guard(ii + 2 < NCH if static_i is not None else i + 2 < NCH,
                      lambda: start_linear(i + 2, d))
                start_scatter(k)

            def triple(i3, carry):
                for k in range(_NB):
                    step(i3 * _NB + k, k)
                return carry
            lax.fori_loop(0, NCH // _NB, triple, 0)
            for i in range(NCH - NCH % _NB, NCH):
                step(i, i % _NB, static_i=i)
            wait_scatter((NCH - 1) % _NB)

            plsc.subcore_barrier()

            def flush_one(r0, p, first):
                pltpu.sync_copy(acc.at[pl.ds(r0, _FC)], fbuf.at[p])
                if add_bias:
                    pltpu.sync_copy(bias.at[pl.ds(r0, _FC)], bbuf)

                    def badd(g, carry):
                        b0 = g * _L
                        bv = bbuf[pl.ds(b0, _L)]
                        for j in range(_L):
                            fbuf[p, b0 + j, :] = fbuf[p, b0 + j, :] + bv[j]
                        return carry
                    lax.fori_loop(0, FG, badd, 0)
                if not first:
                    pltpu.make_async_copy(
                        fbuf.at[p], dst_hbm.at[pl.ds(0, _FC)], fsem.at[p]).wait()
                pltpu.async_copy(fbuf.at[p], dst_hbm.at[pl.ds(coff + r0, _FC)],
                                 fsem.at[p])
                zero_slice(r0)

            def flush_chunk(k, p, first):
                g = s + _NT * k

                @pl.when(g < NF)
                def _():
                    flush_one(g * _FC, p, first)

            flush_chunk(0, 0, True)
            flush_chunk(1, 1, True)

            def fpair(k2, carry):
                k = 2 + k2 * 2
                flush_chunk(k, 0, False)
                flush_chunk(k + 1, 1, False)
                return carry
            lax.fori_loop(0, (FK - 2) // 2, fpair, 0)
            for k in range(2 + 2 * ((FK - 2) // 2), FK):
                flush_chunk(k, k % 2, False)
            for p in range(2):
                pltpu.make_async_copy(
                    fbuf.at[p], dst_hbm.at[pl.ds(0, _FC)], fsem.at[p]).wait()

            plsc.subcore_barrier()

        def zinit(k, carry):
            g = s + _NT * k

            @pl.when(g < NF)
            def _():
                zero_slice(g * _FC)
            return carry
        lax.fori_loop(0, FK, zinit, 0)
        plsc.subcore_barrier()

        spmm(src, 2, t1, add_bias=False)
        spmm(t1, 3, t2, add_bias=True)
        spmm(t2, 2, t1, add_bias=False)
        spmm(t1, 3, out, add_bias=True)

    mesh = plsc.VectorSubcoreMesh(core_axis_name="c", subcore_axis_name="s")
    table = jax.ShapeDtypeStruct((_NC * N, _L), jnp.float32)
    return pl.kernel(
        body,
        out_type=(table, table, table),
        mesh=mesh,
        compiler_params=pltpu.CompilerParams(use_tc_tiling_on_sc=False, needs_layout_passes=False),
        scratch_types=[
            pltpu.VMEM_SHARED((N, _L), jnp.float32),
            pltpu.VMEM((_NB, _EC, _L), jnp.float32),
            pltpu.VMEM((_NB, 4, _EC), jnp.int32),
            pltpu.VMEM((2, _FC, _L), jnp.float32),
            pltpu.VMEM((_ZC, _L), jnp.float32),
            pltpu.VMEM((_FC,), jnp.float32),
            pltpu.SemaphoreType.DMA,
            pltpu.SemaphoreType.DMA((_NB,)),
            pltpu.SemaphoreType.DMA((_NB,)),
            pltpu.SemaphoreType.DMA((2,)),
        ],
    )


def kernel(x, adj_rows, adj_cols, adj_vals, w_vals, bias):
    B, N = x.shape
    E = adj_rows.shape[0]
    fn = _build(N, E, B)
    xsplit = x.reshape(_NC, _L, N).transpose(0, 2, 1).reshape(_NC * N, _L)
    nblk = E // _EC
    cn = (jnp.arange(_NC, dtype=jnp.int32) * N)[:, None]
    cols2 = adj_cols[None, :] + cn
    rows2 = jnp.broadcast_to(adj_rows[None, :], (_NC, E))
    wbits = jnp.broadcast_to(
        lax.bitcast_convert_type(w_vals, jnp.int32)[None, :], (_NC, E))
    abits = jnp.broadcast_to(
        lax.bitcast_convert_type(adj_vals, jnp.int32)[None, :], (_NC, E))
    edata = jnp.stack([cols2, rows2, wbits, abits], axis=1)
    edata = (edata.reshape(_NC, 4, nblk, _EC)
             .transpose(0, 2, 1, 3)
             .reshape(_NC * nblk, 4, _EC))
    out, _, _ = fn(xsplit, edata, bias)
    return out.reshape(_NC, N, _L).transpose(0, 2, 1).reshape(B, N)
